# baseline (speedup 1.0000x reference)
import functools
import math

import jax
import jax.numpy as jnp
from jax import lax
from jax.experimental import pallas as pl
from jax.experimental.pallas import tpu as pltpu
from jax.experimental.pallas import tpu_sc as plsc

N_NODES = 10000
N_EDGES = 320000
DIM = 128
HALF = 64
ONSET = 0.8
CUT = 1.0
LN2 = math.log(2.0)

BE = 2560
NEB = N_EDGES // BE
SPLITS = [(0, 4), (4, 22), (26, 32), (58, 33), (91, 34)]
BN = 400
NNB = N_NODES // BN
BF = 1280
NFB = 10240 // BF
NSUB = 16
KCH = 80
NPAD = 10240
RPT = NPAD // NSUB
RZ = 64


def _softplus(v):
    return jnp.maximum(v, 0.0) + jnp.log1p(jnp.exp(-jnp.abs(v)))


def _edge_stage_body(dist_ref, ef_ref, w1_ref, b1_ref, w2_ref, b2_ref,
                     g_ref):
    d = dist_ref[0, 0, :]
    gamma = DIM / (CUT - 0.0)
    mu = (lax.broadcasted_iota(jnp.int32, (1, DIM), 1).astype(jnp.float32)
          * (CUT / (DIM - 1)))
    bf = jnp.exp(-gamma * (d[:, None] - mu) ** 2)
    h = _softplus(jnp.dot(bf, w1_ref[...], preferred_element_type=jnp.float32)
                  + b1_ref[0, :]) - LN2
    h = _softplus(jnp.dot(h, w2_ref[...], preferred_element_type=jnp.float32)
                  + b2_ref[0, :]) - LN2
    t = (d - ONSET) / (CUT - ONSET)
    ramp = 0.5 * (jnp.cos(jnp.pi * jnp.clip(t, 0.0, 1.0)) + 1.0)
    co = jnp.where(d < ONSET, 1.0, jnp.where(d > CUT, 0.0, ramp))
    g = ef_ref[...] * h * co[:, None]
    gl = jnp.log(jnp.abs(g))
    gs = jnp.where(g < 0.0, 1.0, 0.0)
    g_ref[0, 0] = jnp.concatenate([gl[:, :HALF], gs[:, :HALF]], axis=1)
    g_ref[1, 0] = jnp.concatenate([gl[:, HALF:], gs[:, HALF:]], axis=1)


def _node_stage_body(x_ref, a_ref):
    xv = x_ref[...]
    al = jnp.log(jnp.abs(xv))
    asg = jnp.where(xv < 0.0, 1.0, 0.0)
    a_ref[0, 0] = jnp.concatenate([al[:, :HALF], asg[:, :HALF]], axis=1)
    a_ref[1, 0] = jnp.concatenate([al[:, HALF:], asg[:, HALF:]], axis=1)


def _final_stage_body(h0_ref, h1_ref, h2_ref, h3_ref, h4_ref, w3_ref,
                      b3_ref, out_ref):
    hc = (h0_ref[...] + h1_ref[...] + h2_ref[...] + h3_ref[...]
          + h4_ref[...])
    h = jnp.concatenate([hc[0, :, :HALF], hc[1, :, :HALF]], axis=1)
    sc = jnp.concatenate([hc[0, :, HALF:], hc[1, :, HALF:]], axis=1)
    parity = sc - 2.0 * jnp.floor(sc * 0.5)
    sign = 1.0 - 2.0 * parity
    hv = sign * jnp.exp(h)
    out_ref[...] = _softplus(
        jnp.dot(hv, w3_ref[...], preferred_element_type=jnp.float32)
        + b3_ref[0, :]) - LN2


def _make_sc_scatter_body(nch):
    eps = nch * KCH
    esz = NSUB * eps
    npair = nch // 2

    def body(ipk, a2, g2, zrows,
             acc_out,
             acc_sh, ibuf_a, ibuf_b,
             abuf_a, gbuf_a, abuf_b, gbuf_b,
             sem_aa, sem_ga, sem_ab, sem_gb, sem_s1, sem_s2):
        c = lax.axis_index("c")
        s = lax.axis_index("s")
        base = s * RPT

        pltpu.sync_copy(zrows, abuf_a.at[pl.ds(0, RZ)])

        def zbody(j, carry):
            pltpu.sync_copy(abuf_a.at[pl.ds(0, RZ)],
                            acc_sh.at[pl.ds(base + j * RZ, RZ)])
            return carry

        lax.fori_loop(0, RPT // RZ, zbody, 0)
        plsc.subcore_barrier()

        e0 = s * eps
        lin0 = (c * NSUB + s) * nch

        def load_idx(chunk, ibuf):
            pltpu.sync_copy(ipk.at[lin0 + chunk], ibuf)

        def start_gathers(chunk, ibuf, abuf, gbuf, sem_a, sem_g):
            eoff = e0 + chunk * KCH
            pltpu.async_copy(a2.at[ibuf.at[0]], abuf, sem_a)
            pltpu.async_copy(g2.at[pl.ds(c * esz + eoff, KCH)],
                             gbuf, sem_g)

        def wait_and_scatter(chunk, ibuf, abuf, gbuf, sem_a, sem_g):
            eoff = e0 + chunk * KCH
            pltpu.make_async_copy(a2.at[ibuf.at[0]], abuf, sem_a).wait()
            pltpu.make_async_copy(g2.at[pl.ds(c * esz + eoff, KCH)],
                                  gbuf, sem_g).wait()
            ca = pltpu.async_copy(abuf, acc_sh.at[ibuf.at[1]], sem_s1,
                                  add=True)
            cg = pltpu.async_copy(gbuf, acc_sh.at[ibuf.at[1]], sem_s2,
                                  add=True)
            ca.wait()
            cg.wait()

        load_idx(0, ibuf_a)
        start_gathers(0, ibuf_a, abuf_a, gbuf_a, sem_aa, sem_ga)

        def ebody(i, carry):
            ca = 2 * i
            cb = 2 * i + 1
            load_idx(cb, ibuf_b)
            start_gathers(cb, ibuf_b, abuf_b, gbuf_b, sem_ab, sem_gb)
            wait_and_scatter(ca, ibuf_a, abuf_a, gbuf_a, sem_aa, sem_ga)

            @pl.when(i < npair - 1)
            def _():
                load_idx(ca + 2, ibuf_a)
                start_gathers(ca + 2, ibuf_a, abuf_a, gbuf_a, sem_aa,
                              sem_ga)

            wait_and_scatter(cb, ibuf_b, abuf_b, gbuf_b, sem_ab, sem_gb)
            return carry

        lax.fori_loop(0, npair, ebody, 0)
        plsc.subcore_barrier()

        def obody(j, carry):
            r0 = base + j * RZ
            pltpu.sync_copy(acc_sh.at[pl.ds(r0, RZ)],
                            abuf_a.at[pl.ds(0, RZ)])
            pltpu.sync_copy(abuf_a.at[pl.ds(0, RZ)],
                            acc_out.at[pl.ds(c * NPAD + r0, RZ)])
            return carry

        lax.fori_loop(0, RPT // RZ, obody, 0)

    return body


def _make_edge_stage(blk0, nb):
    return pl.pallas_call(
        _edge_stage_body,
        grid=(nb,),
        in_specs=[
            pl.BlockSpec((1, 1, BE), lambda i: (blk0 + i, 0, 0)),
            pl.BlockSpec((BE, DIM), lambda i: (blk0 + i, 0)),
            pl.BlockSpec((DIM, DIM), lambda i: (0, 0)),
            pl.BlockSpec((1, DIM), lambda i: (0, 0)),
            pl.BlockSpec((DIM, DIM), lambda i: (0, 0)),
            pl.BlockSpec((1, DIM), lambda i: (0, 0)),
        ],
        out_specs=pl.BlockSpec((2, 1, BE, DIM), lambda i: (0, i, 0, 0)),
        out_shape=jax.ShapeDtypeStruct((2, nb, BE, DIM), jnp.float32),
    )


def _make_sc_scatter(nch):
    return functools.partial(
        pl.kernel,
        mesh=plsc.VectorSubcoreMesh(core_axis_name="c", subcore_axis_name="s"),
        compiler_params=pltpu.CompilerParams(use_tc_tiling_on_sc=False),
        out_type=jax.ShapeDtypeStruct((2 * NPAD, DIM), jnp.float32),
        scratch_types=[
            pltpu.VMEM_SHARED((NPAD, DIM), jnp.float32),
            pltpu.VMEM((2, KCH), jnp.int32),
            pltpu.VMEM((2, KCH), jnp.int32),
            pltpu.VMEM((KCH, DIM), jnp.float32),
            pltpu.VMEM((KCH, DIM), jnp.float32),
            pltpu.VMEM((KCH, DIM), jnp.float32),
            pltpu.VMEM((KCH, DIM), jnp.float32),
            pltpu.SemaphoreType.DMA,
            pltpu.SemaphoreType.DMA,
            pltpu.SemaphoreType.DMA,
            pltpu.SemaphoreType.DMA,
            pltpu.SemaphoreType.DMA,
            pltpu.SemaphoreType.DMA,
        ],
    )(_make_sc_scatter_body(nch))


_edge_stages = [_make_edge_stage(b0, nb) for b0, nb in SPLITS]
_sc_scatters = [_make_sc_scatter(nb * BE // (NSUB * KCH)) for _, nb in SPLITS]

_node_stage = pl.pallas_call(
    _node_stage_body,
    grid=(NNB,),
    in_specs=[pl.BlockSpec((BN, DIM), lambda i: (i, 0))],
    out_specs=pl.BlockSpec((2, 1, BN, DIM), lambda i: (0, i, 0, 0)),
    out_shape=jax.ShapeDtypeStruct((2, NNB, BN, DIM), jnp.float32),
)

_final_stage = pl.pallas_call(
    _final_stage_body,
    grid=(NFB,),
    in_specs=[
        pl.BlockSpec((2, BF, DIM), lambda i: (0, i, 0)),
        pl.BlockSpec((2, BF, DIM), lambda i: (0, i, 0)),
        pl.BlockSpec((2, BF, DIM), lambda i: (0, i, 0)),
        pl.BlockSpec((2, BF, DIM), lambda i: (0, i, 0)),
        pl.BlockSpec((2, BF, DIM), lambda i: (0, i, 0)),
        pl.BlockSpec((DIM, DIM), lambda i: (0, 0)),
        pl.BlockSpec((1, DIM), lambda i: (0, 0)),
    ],
    out_specs=pl.BlockSpec((BF, DIM), lambda i: (i, 0)),
    out_shape=jax.ShapeDtypeStruct((NPAD, DIM), jnp.float32),
)


def _pack_idx(srch, dsth, nch):
    srcr = srch.reshape(NSUB * nch, KCH)
    dstr = dsth.reshape(NSUB * nch, KCH)
    return jnp.concatenate(
        [jnp.stack([srcr, dstr], axis=1),
         jnp.stack([srcr + N_NODES, dstr], axis=1)], axis=0)


def kernel(x, edge_index, edge_feat, dist, W1, b1, W2, b2, W3, b3):
    src = edge_index[0].astype(jnp.int32)
    dst = edge_index[1].astype(jnp.int32)
    dist3 = dist.reshape(NEB, 1, BE)
    b1r = b1.reshape(1, DIM)
    b2r = b2.reshape(1, DIM)
    b3r = b3.reshape(1, DIM)

    a4 = _node_stage(x)
    a2 = a4.reshape(2 * N_NODES, DIM)
    zrows = jnp.zeros((RZ, DIM), jnp.float32)

    accs = []
    for k, (b0, nb) in enumerate(SPLITS):
        e0 = b0 * BE
        esz = nb * BE
        nch = esz // (NSUB * KCH)
        ipk = _pack_idx(src[e0:e0 + esz], dst[e0:e0 + esz], nch)
        g4 = _edge_stages[k](dist3, edge_feat, W1, b1r, W2, b2r)
        g2r = g4.reshape(2 * esz, DIM)
        if accs:
            ipk, accs[-1] = lax.optimization_barrier((ipk, accs[-1]))
        accs.append(_sc_scatters[k](ipk, a2, g2r, zrows))

    out = _final_stage(*[a.reshape(2, NPAD, DIM) for a in accs], W3, b3r)
    return out[:N_NODES]


# speedup vs baseline: 1.0095x; 1.0095x over previous
import functools
import math

import jax
import jax.numpy as jnp
from jax import lax
from jax.experimental import pallas as pl
from jax.experimental.pallas import tpu as pltpu
from jax.experimental.pallas import tpu_sc as plsc

N_NODES = 10000
N_EDGES = 320000
DIM = 128
HALF = 64
ONSET = 0.8
CUT = 1.0
LN2 = math.log(2.0)

BE = 2560
NEB = N_EDGES // BE
SPLITS = [(0, 10), (10, 20), (30, 30), (60, 32), (92, 33)]
BN = 400
NNB = N_NODES // BN
BF = 1280
NFB = 10240 // BF
NSUB = 16
KCH = 80
NPAD = 10240
RPT = NPAD // NSUB
RZ = 64


def _softplus(v):
    return jnp.maximum(v, 0.0) + jnp.log1p(jnp.exp(-jnp.abs(v)))


def _edge_stage_body(dist_ref, ef_ref, w1_ref, b1_ref, w2_ref, b2_ref,
                     g_ref):
    d = dist_ref[0, 0, :]
    gamma = DIM / (CUT - 0.0)
    mu = (lax.broadcasted_iota(jnp.int32, (1, DIM), 1).astype(jnp.float32)
          * (CUT / (DIM - 1)))
    bf = jnp.exp(-gamma * (d[:, None] - mu) ** 2)
    h = _softplus(jnp.dot(bf, w1_ref[...], preferred_element_type=jnp.float32)
                  + b1_ref[0, :]) - LN2
    h = _softplus(jnp.dot(h, w2_ref[...], preferred_element_type=jnp.float32)
                  + b2_ref[0, :]) - LN2
    t = (d - ONSET) / (CUT - ONSET)
    ramp = 0.5 * (jnp.cos(jnp.pi * jnp.clip(t, 0.0, 1.0)) + 1.0)
    co = jnp.where(d < ONSET, 1.0, jnp.where(d > CUT, 0.0, ramp))
    g = ef_ref[...] * h * co[:, None]
    gl = jnp.log(jnp.abs(g))
    gs = jnp.where(g < 0.0, 1.0, 0.0)
    g_ref[0, 0] = jnp.concatenate([gl[:, :HALF], gs[:, :HALF]], axis=1)
    g_ref[1, 0] = jnp.concatenate([gl[:, HALF:], gs[:, HALF:]], axis=1)


def _node_stage_body(x_ref, a_ref):
    xv = x_ref[...]
    al = jnp.log(jnp.abs(xv))
    asg = jnp.where(xv < 0.0, 1.0, 0.0)
    a_ref[0, 0] = jnp.concatenate([al[:, :HALF], asg[:, :HALF]], axis=1)
    a_ref[1, 0] = jnp.concatenate([al[:, HALF:], asg[:, HALF:]], axis=1)


def _final_stage_body(h0_ref, h1_ref, h2_ref, h3_ref, h4_ref, w3_ref,
                      b3_ref, out_ref):
    hc = (h0_ref[...] + h1_ref[...] + h2_ref[...] + h3_ref[...]
          + h4_ref[...])
    h = jnp.concatenate([hc[0, :, :HALF], hc[1, :, :HALF]], axis=1)
    sc = jnp.concatenate([hc[0, :, HALF:], hc[1, :, HALF:]], axis=1)
    parity = sc - 2.0 * jnp.floor(sc * 0.5)
    sign = 1.0 - 2.0 * parity
    hv = sign * jnp.exp(h)
    out_ref[...] = _softplus(
        jnp.dot(hv, w3_ref[...], preferred_element_type=jnp.float32)
        + b3_ref[0, :]) - LN2


def _make_sc_scatter_body(nch):
    eps = nch * KCH
    esz = NSUB * eps
    npair = nch // 2

    def body(ipk, a2, g2, zrows,
             acc_out,
             acc_sh, ibuf_a, ibuf_b,
             abuf_a, gbuf_a, abuf_b, gbuf_b,
             sem_aa, sem_ga, sem_ab, sem_gb, sem_s1, sem_s2):
        c = lax.axis_index("c")
        s = lax.axis_index("s")
        base = s * RPT

        pltpu.sync_copy(zrows, abuf_a.at[pl.ds(0, RZ)])

        def zbody(j, carry):
            pltpu.sync_copy(abuf_a.at[pl.ds(0, RZ)],
                            acc_sh.at[pl.ds(base + j * RZ, RZ)])
            return carry

        lax.fori_loop(0, RPT // RZ, zbody, 0)
        plsc.subcore_barrier()

        e0 = s * eps
        lin0 = (c * NSUB + s) * nch

        def load_idx(chunk, ibuf):
            pltpu.sync_copy(ipk.at[lin0 + chunk], ibuf)

        def start_gathers(chunk, ibuf, abuf, gbuf, sem_a, sem_g):
            eoff = e0 + chunk * KCH
            pltpu.async_copy(a2.at[ibuf.at[0]], abuf, sem_a)
            pltpu.async_copy(g2.at[pl.ds(c * esz + eoff, KCH)],
                             gbuf, sem_g)

        def wait_and_scatter(chunk, ibuf, abuf, gbuf, sem_a, sem_g):
            eoff = e0 + chunk * KCH
            pltpu.make_async_copy(a2.at[ibuf.at[0]], abuf, sem_a).wait()
            pltpu.make_async_copy(g2.at[pl.ds(c * esz + eoff, KCH)],
                                  gbuf, sem_g).wait()
            ca = pltpu.async_copy(abuf, acc_sh.at[ibuf.at[1]], sem_s1,
                                  add=True)
            cg = pltpu.async_copy(gbuf, acc_sh.at[ibuf.at[1]], sem_s2,
                                  add=True)
            ca.wait()
            cg.wait()

        load_idx(0, ibuf_a)
        start_gathers(0, ibuf_a, abuf_a, gbuf_a, sem_aa, sem_ga)

        def ebody(i, carry):
            ca = 2 * i
            cb = 2 * i + 1
            load_idx(cb, ibuf_b)
            start_gathers(cb, ibuf_b, abuf_b, gbuf_b, sem_ab, sem_gb)
            wait_and_scatter(ca, ibuf_a, abuf_a, gbuf_a, sem_aa, sem_ga)

            @pl.when(i < npair - 1)
            def _():
                load_idx(ca + 2, ibuf_a)
                start_gathers(ca + 2, ibuf_a, abuf_a, gbuf_a, sem_aa,
                              sem_ga)

            wait_and_scatter(cb, ibuf_b, abuf_b, gbuf_b, sem_ab, sem_gb)
            return carry

        lax.fori_loop(0, npair, ebody, 0)
        plsc.subcore_barrier()

        def obody(j, carry):
            r0 = base + j * RZ
            pltpu.sync_copy(acc_sh.at[pl.ds(r0, RZ)],
                            abuf_a.at[pl.ds(0, RZ)])
            pltpu.sync_copy(abuf_a.at[pl.ds(0, RZ)],
                            acc_out.at[pl.ds(c * NPAD + r0, RZ)])
            return carry

        lax.fori_loop(0, RPT // RZ, obody, 0)

    return body


def _make_edge_stage(blk0, nb):
    return pl.pallas_call(
        _edge_stage_body,
        grid=(nb,),
        in_specs=[
            pl.BlockSpec((1, 1, BE), lambda i: (blk0 + i, 0, 0)),
            pl.BlockSpec((BE, DIM), lambda i: (blk0 + i, 0)),
            pl.BlockSpec((DIM, DIM), lambda i: (0, 0)),
            pl.BlockSpec((1, DIM), lambda i: (0, 0)),
            pl.BlockSpec((DIM, DIM), lambda i: (0, 0)),
            pl.BlockSpec((1, DIM), lambda i: (0, 0)),
        ],
        out_specs=pl.BlockSpec((2, 1, BE, DIM), lambda i: (0, i, 0, 0)),
        out_shape=jax.ShapeDtypeStruct((2, nb, BE, DIM), jnp.float32),
    )


def _make_sc_scatter(nch):
    return functools.partial(
        pl.kernel,
        mesh=plsc.VectorSubcoreMesh(core_axis_name="c", subcore_axis_name="s"),
        compiler_params=pltpu.CompilerParams(use_tc_tiling_on_sc=False),
        out_type=jax.ShapeDtypeStruct((2 * NPAD, DIM), jnp.float32),
        scratch_types=[
            pltpu.VMEM_SHARED((NPAD, DIM), jnp.float32),
            pltpu.VMEM((2, KCH), jnp.int32),
            pltpu.VMEM((2, KCH), jnp.int32),
            pltpu.VMEM((KCH, DIM), jnp.float32),
            pltpu.VMEM((KCH, DIM), jnp.float32),
            pltpu.VMEM((KCH, DIM), jnp.float32),
            pltpu.VMEM((KCH, DIM), jnp.float32),
            pltpu.SemaphoreType.DMA,
            pltpu.SemaphoreType.DMA,
            pltpu.SemaphoreType.DMA,
            pltpu.SemaphoreType.DMA,
            pltpu.SemaphoreType.DMA,
            pltpu.SemaphoreType.DMA,
        ],
    )(_make_sc_scatter_body(nch))


_edge_stages = [_make_edge_stage(b0, nb) for b0, nb in SPLITS]
_sc_scatters = [_make_sc_scatter(nb * BE // (NSUB * KCH)) for _, nb in SPLITS]

_node_stage = pl.pallas_call(
    _node_stage_body,
    grid=(NNB,),
    in_specs=[pl.BlockSpec((BN, DIM), lambda i: (i, 0))],
    out_specs=pl.BlockSpec((2, 1, BN, DIM), lambda i: (0, i, 0, 0)),
    out_shape=jax.ShapeDtypeStruct((2, NNB, BN, DIM), jnp.float32),
)

_final_stage = pl.pallas_call(
    _final_stage_body,
    grid=(NFB,),
    in_specs=[
        pl.BlockSpec((2, BF, DIM), lambda i: (0, i, 0)),
        pl.BlockSpec((2, BF, DIM), lambda i: (0, i, 0)),
        pl.BlockSpec((2, BF, DIM), lambda i: (0, i, 0)),
        pl.BlockSpec((2, BF, DIM), lambda i: (0, i, 0)),
        pl.BlockSpec((2, BF, DIM), lambda i: (0, i, 0)),
        pl.BlockSpec((DIM, DIM), lambda i: (0, 0)),
        pl.BlockSpec((1, DIM), lambda i: (0, 0)),
    ],
    out_specs=pl.BlockSpec((BF, DIM), lambda i: (i, 0)),
    out_shape=jax.ShapeDtypeStruct((NPAD, DIM), jnp.float32),
)


def _pack_idx(srch, dsth, nch):
    srcr = srch.reshape(NSUB * nch, KCH)
    dstr = dsth.reshape(NSUB * nch, KCH)
    return jnp.concatenate(
        [jnp.stack([srcr, dstr], axis=1),
         jnp.stack([srcr + N_NODES, dstr], axis=1)], axis=0)


def kernel(x, edge_index, edge_feat, dist, W1, b1, W2, b2, W3, b3):
    src = edge_index[0].astype(jnp.int32)
    dst = edge_index[1].astype(jnp.int32)
    dist3 = dist.reshape(NEB, 1, BE)
    b1r = b1.reshape(1, DIM)
    b2r = b2.reshape(1, DIM)
    b3r = b3.reshape(1, DIM)

    a4 = _node_stage(x)
    a2 = a4.reshape(2 * N_NODES, DIM)
    zrows = jnp.zeros((RZ, DIM), jnp.float32)

    accs = []
    for k, (b0, nb) in enumerate(SPLITS):
        e0 = b0 * BE
        esz = nb * BE
        nch = esz // (NSUB * KCH)
        ipk = _pack_idx(src[e0:e0 + esz], dst[e0:e0 + esz], nch)
        g4 = _edge_stages[k](dist3, edge_feat, W1, b1r, W2, b2r)
        g2r = g4.reshape(2 * esz, DIM)
        if accs:
            ipk, accs[-1] = lax.optimization_barrier((ipk, accs[-1]))
        accs.append(_sc_scatters[k](ipk, a2, g2r, zrows))

    out = _final_stage(*[a.reshape(2, NPAD, DIM) for a in accs], W3, b3r)
    return out[:N_NODES]
